# baseline (speedup 1.0000x reference)
import jax
import jax.numpy as jnp
from jax import lax
from jax.experimental import pallas as pl
from jax.experimental.pallas import tpu as pltpu
from jax.experimental.pallas import tpu_sc as plsc

NC, NS = 2, 16
NW = NC * NS
CHUNK = 32


def _sc_body(table_hbm, out_hbm, buf):
    wid = lax.axis_index("s") * NC + lax.axis_index("c")
    rows_per_w = table_hbm.shape[0] // NW
    base = wid * rows_per_w

    def step(i, carry):
        row = base + i * CHUNK
        pltpu.sync_copy(table_hbm.at[pl.ds(row, CHUNK)], buf)
        for b in range(4):
            pltpu.sync_copy(buf, out_hbm.at[b, pl.ds(row, CHUNK)])
        return carry

    lax.fori_loop(0, rows_per_w // CHUNK, step, 0)


def kernel(B, T, pos_weight):
    t_static, d = pos_weight.shape
    run = pl.kernel(
        _sc_body,
        out_type=jax.ShapeDtypeStruct((4, t_static, d), pos_weight.dtype),
        mesh=plsc.VectorSubcoreMesh(core_axis_name="c", subcore_axis_name="s"),
        scratch_types=[
            pltpu.VMEM((CHUNK, d), jnp.float32),
        ],
    )
    return run(pos_weight)
